# Initial kernel scaffold; baseline (speedup 1.0000x reference)
import jax
import jax.numpy as jnp
from jax.experimental import pallas as pl


def kernel(x, a, e, Ws1, bs1, Wai1, bai1, Wao1, bao1, Wn1, bn1, We1, be1, Ws2, bs2, Wai2, bai2, Wao2, bao2, Wn2, bn2, We2, be2, Wd, bd):
    raise NotImplementedError("write your pallas kernel here")


# speedup vs baseline: 3.8576x; 3.8576x over previous
import functools

import jax
import jax.numpy as jnp
from jax.experimental import pallas as pl

N = 512
BI = 128
STACK = 32


def _edge_layer_kernel(x_ref, a_ref, e_ref, et_ref, wsx_ref, wsv_ref,
                       we_ref, wet_ref, bs_ref, wai_ref, bai_ref,
                       wao_ref, bao_ref, wedge_ref, bedge_ref,
                       *refs, emit_edge):
    if emit_edge:
        e1_ref, e1t_ref, min_ref, mout_ref = refs
    else:
        min_ref, mout_ref = refs
    i = pl.program_id(0)
    u = jnp.dot(x_ref[pl.ds(i * BI, BI), :], wsx_ref[...],
                preferred_element_type=jnp.float32)
    vt = jax.lax.dot_general(wsv_ref[...], x_ref[...], (((0,), (1,)), ((), ())),
                             preferred_element_type=jnp.float32)
    e_blk = e_ref[...]
    et_blk = et_ref[...]
    we = we_ref[...].reshape(1, STACK, 1)
    wet = wet_ref[...].reshape(1, STACK, 1)
    bs = bs_ref[...].reshape(1, STACK, 1)
    t = u[:, :, None] + vt[None, :, :] \
        + e_blk[:, None, :] * we + et_blk[:, None, :] * wet + bs
    t = jnp.maximum(t, 0.0)

    wai = wai_ref[...].reshape(1, STACK, 1)
    wao = wao_ref[...].reshape(1, STACK, 1)
    att_i = jax.nn.sigmoid(jnp.sum(t * wai, axis=1) + bai_ref[0, 0])
    att_j = jax.nn.sigmoid(jnp.sum(t * wao, axis=1) + bao_ref[0, 0])
    mask = (a_ref[...] != 0.0).astype(jnp.float32)

    if emit_edge:
        wedge = wedge_ref[...].reshape(1, STACK, 1)
        e1 = jnp.sum(t * wedge, axis=1) + bedge_ref[0, 0]
        e1_ref[...] = e1
        e1t_ref[...] = e1.T

    w_in = mask * att_i
    w_out = mask * att_j
    min_ref[...] = jnp.sum(t * w_in[:, None, :], axis=2)
    part_out = jnp.sum(t * w_out[:, None, :], axis=0)

    @pl.when(i == 0)
    def _init():
        mout_ref[...] = jnp.zeros_like(mout_ref)

    mout_ref[...] += part_out


def _edge_layer(x, a, e, et, Ws, bs, Wai, bai, Wao, bao, We, be, emit_edge):
    f = x.shape[1]
    wsx = Ws[:f, :]
    wsv = Ws[f:2 * f, :]
    we = Ws[2 * f:2 * f + 1, :]
    wet = Ws[2 * f + 1:, :]
    bs2 = bs.reshape(1, STACK)
    wai = Wai.reshape(1, STACK)
    wao = Wao.reshape(1, STACK)
    wedge = We.reshape(1, STACK)
    bai2 = bai.reshape(1, 1)
    bao2 = bao.reshape(1, 1)
    be2 = be.reshape(1, 1)

    grid = (N // BI,)
    full = lambda s: pl.BlockSpec(s, lambda i: (0,) * len(s))
    row_blk = pl.BlockSpec((BI, N), lambda i: (i, 0))
    out_shapes = [
        jax.ShapeDtypeStruct((N, STACK), jnp.float32),
        jax.ShapeDtypeStruct((STACK, N), jnp.float32),
    ]
    out_specs = [
        pl.BlockSpec((BI, STACK), lambda i: (i, 0)),
        full((STACK, N)),
    ]
    if emit_edge:
        out_shapes = [jax.ShapeDtypeStruct((N, N), jnp.float32),
                      jax.ShapeDtypeStruct((N, N), jnp.float32)] + out_shapes
        out_specs = [row_blk,
                     pl.BlockSpec((N, BI), lambda i: (0, i))] + out_specs
    outs = pl.pallas_call(
        functools.partial(_edge_layer_kernel, emit_edge=emit_edge),
        grid=grid,
        in_specs=[
            full((N, f)),
            row_blk,
            row_blk,
            row_blk,
            full((f, STACK)),
            full((f, STACK)),
            full((1, STACK)),
            full((1, STACK)),
            full((1, STACK)),
            full((1, STACK)),
            full((1, 1)),
            full((1, STACK)),
            full((1, 1)),
            full((1, STACK)),
            full((1, 1)),
        ],
        out_shape=out_shapes,
        out_specs=out_specs,
    )(x, a, e, et, wsx, wsv, we, wet, bs2, wai, bai2, wao, bao2, wedge, be2)
    if emit_edge:
        return outs
    return (None, None) + tuple(outs)


def _node_update_kernel(x_ref, min_ref, mout_ref, wna_ref, wnb_ref, wnc_ref,
                        bn_ref, wd_ref, bd_ref, out_ref, *, readout):
    xn = jnp.dot(x_ref[...], wna_ref[...], preferred_element_type=jnp.float32)
    xn += jnp.dot(min_ref[...], wnb_ref[...],
                  preferred_element_type=jnp.float32)
    xn += jax.lax.dot_general(mout_ref[...], wnc_ref[...],
                              (((0,), (0,)), ((), ())),
                              preferred_element_type=jnp.float32)
    xn += bn_ref[...]
    if readout:
        xn = jnp.dot(xn, wd_ref[...],
                     preferred_element_type=jnp.float32) + bd_ref[...]
    out_ref[...] = xn


def _node_update(x, m_in, m_out, Wn, bn, Wd, bd, readout):
    f = x.shape[1]
    wna = Wn[:f, :]
    wnb = Wn[f:f + STACK, :]
    wnc = Wn[f + STACK:, :]
    node = Wn.shape[1]
    if readout:
        nout = Wd.shape[1]
    else:
        nout = node
        Wd = jnp.zeros((1, 1), jnp.float32)
        bd = jnp.zeros((1,), jnp.float32)
    full = lambda s: pl.BlockSpec(s, lambda: (0,) * len(s))
    return pl.pallas_call(
        functools.partial(_node_update_kernel, readout=readout),
        in_specs=[full((N, f)), full((N, STACK)), full((STACK, N)),
                  full((f, node)), full((STACK, node)), full((STACK, node)),
                  full((1, node)), full(Wd.shape), full((1, bd.shape[0]))],
        out_shape=jax.ShapeDtypeStruct((N, nout), jnp.float32),
        out_specs=full((N, nout)),
    )(x, m_in, m_out, wna, wnb, wnc, bn.reshape(1, node), Wd,
      bd.reshape(1, -1))


def kernel(x, a, e, Ws1, bs1, Wai1, bai1, Wao1, bao1, Wn1, bn1, We1, be1,
           Ws2, bs2, Wai2, bai2, Wao2, bao2, Wn2, bn2, We2, be2, Wd, bd):
    x2d = x[0]
    a2d = a[0]
    e2d = e[0, :, :, 0]
    et2d = e2d.T

    e1, e1t, m_in1, m_out1 = _edge_layer(
        x2d, a2d, e2d, et2d, Ws1, bs1, Wai1, bai1, Wao1, bao1, We1, be1,
        emit_edge=True)
    x1 = _node_update(x2d, m_in1, m_out1, Wn1, bn1, Wd, bd, readout=False)
    _, _, m_in2, m_out2 = _edge_layer(
        x1, a2d, e1, e1t, Ws2, bs2, Wai2, bai2, Wao2, bao2, We2, be2,
        emit_edge=False)
    out = _node_update(x1, m_in2, m_out2, Wn2, bn2, Wd, bd, readout=True)
    return out[None, :, :]


# speedup vs baseline: 4.4764x; 1.1604x over previous
import functools

import jax
import jax.numpy as jnp
from jax.experimental import pallas as pl

N = 512
BI = 128
STACK = 32


def _edge_layer_kernel(x_ref, a_ref, e_ref, et_ref, wsx_ref, wsv_ref,
                       we_ref, wet_ref, bs_ref, w3_ref, bai_ref,
                       bao_ref, bedge_ref,
                       *refs, emit_edge):
    if emit_edge:
        e1_ref, e1t_ref, min_ref, mout_ref = refs
    else:
        min_ref, mout_ref = refs
    i = pl.program_id(0)
    u = jnp.dot(x_ref[pl.ds(i * BI, BI), :], wsx_ref[...],
                preferred_element_type=jnp.float32) + bs_ref[...]
    vt = jax.lax.dot_general(wsv_ref[...], x_ref[...], (((0,), (1,)), ((), ())),
                             preferred_element_type=jnp.float32)
    e_blk = e_ref[...]
    et_blk = et_ref[...]
    we = we_ref[...].reshape(1, STACK, 1)
    wet = wet_ref[...].reshape(1, STACK, 1)
    t = u[:, :, None] + vt[None, :, :] \
        + e_blk[:, None, :] * we + et_blk[:, None, :] * wet
    t = jnp.maximum(t, 0.0)

    k = w3_ref.shape[0]
    w3b = jnp.broadcast_to(w3_ref[...][None, :, :], (BI, k, STACK))
    proj = jax.lax.dot_general(w3b, t, (((2,), (1,)), ((0,), (0,))),
                               preferred_element_type=jnp.float32)
    att_i = jax.nn.sigmoid(proj[:, 0, :] + bai_ref[0, 0])
    att_j = jax.nn.sigmoid(proj[:, 1, :] + bao_ref[0, 0])
    mask = (a_ref[...] != 0.0).astype(jnp.float32)

    if emit_edge:
        e1 = proj[:, 2, :] + bedge_ref[0, 0]
        e1_ref[...] = e1
        e1t_ref[...] = e1.T

    w_in = mask * att_i
    w_out = mask * att_j
    min_ref[...] = jax.lax.dot_general(t, w_in, (((2,), (1,)), ((0,), (0,))),
                                       preferred_element_type=jnp.float32)
    part_out = jnp.sum(t * w_out[:, None, :], axis=0)

    @pl.when(i == 0)
    def _init():
        mout_ref[...] = jnp.zeros_like(mout_ref)

    mout_ref[...] += part_out


def _edge_layer(x, a, e, et, Ws, bs, Wai, bai, Wao, bao, We, be, emit_edge):
    f = x.shape[1]
    wsx = Ws[:f, :]
    wsv = Ws[f:2 * f, :]
    we = Ws[2 * f:2 * f + 1, :]
    wet = Ws[2 * f + 1:, :]
    bs2 = bs.reshape(1, STACK)
    rows = [Wai.reshape(1, STACK), Wao.reshape(1, STACK)]
    if emit_edge:
        rows.append(We.reshape(1, STACK))
    w3 = jnp.concatenate(rows, axis=0)
    bai2 = bai.reshape(1, 1)
    bao2 = bao.reshape(1, 1)
    be2 = be.reshape(1, 1)

    grid = (N // BI,)
    full = lambda s: pl.BlockSpec(s, lambda i: (0,) * len(s))
    row_blk = pl.BlockSpec((BI, N), lambda i: (i, 0))
    out_shapes = [
        jax.ShapeDtypeStruct((N, STACK), jnp.float32),
        jax.ShapeDtypeStruct((STACK, N), jnp.float32),
    ]
    out_specs = [
        pl.BlockSpec((BI, STACK), lambda i: (i, 0)),
        full((STACK, N)),
    ]
    if emit_edge:
        out_shapes = [jax.ShapeDtypeStruct((N, N), jnp.float32),
                      jax.ShapeDtypeStruct((N, N), jnp.float32)] + out_shapes
        out_specs = [row_blk,
                     pl.BlockSpec((N, BI), lambda i: (0, i))] + out_specs
    outs = pl.pallas_call(
        functools.partial(_edge_layer_kernel, emit_edge=emit_edge),
        grid=grid,
        in_specs=[
            full((N, f)),
            row_blk,
            row_blk,
            row_blk,
            full((f, STACK)),
            full((f, STACK)),
            full((1, STACK)),
            full((1, STACK)),
            full((1, STACK)),
            full(w3.shape),
            full((1, 1)),
            full((1, 1)),
            full((1, 1)),
        ],
        out_shape=out_shapes,
        out_specs=out_specs,
    )(x, a, e, et, wsx, wsv, we, wet, bs2, w3, bai2, bao2, be2)
    if emit_edge:
        return outs
    return (None, None) + tuple(outs)


def _node_update_kernel(x_ref, min_ref, mout_ref, wna_ref, wnb_ref, wnc_ref,
                        bn_ref, wd_ref, bd_ref, out_ref, *, readout):
    xn = jnp.dot(x_ref[...], wna_ref[...], preferred_element_type=jnp.float32)
    xn += jnp.dot(min_ref[...], wnb_ref[...],
                  preferred_element_type=jnp.float32)
    xn += jax.lax.dot_general(mout_ref[...], wnc_ref[...],
                              (((0,), (0,)), ((), ())),
                              preferred_element_type=jnp.float32)
    xn += bn_ref[...]
    if readout:
        xn = jnp.dot(xn, wd_ref[...],
                     preferred_element_type=jnp.float32) + bd_ref[...]
    out_ref[...] = xn


def _node_update(x, m_in, m_out, Wn, bn, Wd, bd, readout):
    f = x.shape[1]
    wna = Wn[:f, :]
    wnb = Wn[f:f + STACK, :]
    wnc = Wn[f + STACK:, :]
    node = Wn.shape[1]
    if readout:
        nout = Wd.shape[1]
    else:
        nout = node
        Wd = jnp.zeros((1, 1), jnp.float32)
        bd = jnp.zeros((1,), jnp.float32)
    full = lambda s: pl.BlockSpec(s, lambda: (0,) * len(s))
    return pl.pallas_call(
        functools.partial(_node_update_kernel, readout=readout),
        in_specs=[full((N, f)), full((N, STACK)), full((STACK, N)),
                  full((f, node)), full((STACK, node)), full((STACK, node)),
                  full((1, node)), full(Wd.shape), full((1, bd.shape[0]))],
        out_shape=jax.ShapeDtypeStruct((N, nout), jnp.float32),
        out_specs=full((N, nout)),
    )(x, m_in, m_out, wna, wnb, wnc, bn.reshape(1, node), Wd,
      bd.reshape(1, -1))


def kernel(x, a, e, Ws1, bs1, Wai1, bai1, Wao1, bao1, Wn1, bn1, We1, be1,
           Ws2, bs2, Wai2, bai2, Wao2, bao2, Wn2, bn2, We2, be2, Wd, bd):
    x2d = x[0]
    a2d = a[0]
    e2d = e[0, :, :, 0]
    et2d = e2d.T

    e1, e1t, m_in1, m_out1 = _edge_layer(
        x2d, a2d, e2d, et2d, Ws1, bs1, Wai1, bai1, Wao1, bao1, We1, be1,
        emit_edge=True)
    x1 = _node_update(x2d, m_in1, m_out1, Wn1, bn1, Wd, bd, readout=False)
    _, _, m_in2, m_out2 = _edge_layer(
        x1, a2d, e1, e1t, Ws2, bs2, Wai2, bai2, Wao2, bao2, We2, be2,
        emit_edge=False)
    out = _node_update(x1, m_in2, m_out2, Wn2, bn2, Wd, bd, readout=True)
    return out[None, :, :]


# speedup vs baseline: 4.8983x; 1.0942x over previous
import jax
import jax.numpy as jnp
from jax.experimental import pallas as pl
from jax.experimental.pallas import tpu as pltpu

N = 512
BI = 128
STACK = 32
F32 = jnp.float32


def _edge_step(idx, x_ref, edge_ref, a_ref, wsx, wsv, bs, we, wet, w3, bvec,
               e1_scr, min_scr, mout_scr, emit_edge):
    rows = pl.ds(idx * BI, BI)
    u = jnp.dot(x_ref[rows, :], wsx, preferred_element_type=F32) + bs
    vt = jax.lax.dot_general(wsv, x_ref[...], (((0,), (1,)), ((), ())),
                             preferred_element_type=F32)
    e_blk = edge_ref[rows, :]
    et_blk = edge_ref[:, rows].T
    t = u[:, :, None] + vt[None, :, :] \
        + e_blk[:, None, :] * we.reshape(1, STACK, 1) \
        + et_blk[:, None, :] * wet.reshape(1, STACK, 1)
    t = jnp.maximum(t, 0.0)

    k = w3.shape[0]
    w3b = jnp.broadcast_to(w3[None, :, :], (BI, k, STACK))
    proj = jax.lax.dot_general(w3b, t, (((2,), (1,)), ((0,), (0,))),
                               preferred_element_type=F32)
    att_i = jax.nn.sigmoid(proj[:, 0, :] + bvec[0, 0])
    att_j = jax.nn.sigmoid(proj[:, 1, :] + bvec[0, 1])
    mask = (a_ref[rows, :] != 0.0).astype(F32)
    if emit_edge:
        e1_scr[rows, :] = proj[:, 2, :] + bvec[0, 2]
    w_in = mask * att_i
    w_out = mask * att_j
    min_scr[rows, :] = jax.lax.dot_general(
        t, w_in, (((2,), (1,)), ((0,), (0,))), preferred_element_type=F32)
    mout_scr[...] += jnp.sum(t * w_out[:, None, :], axis=0)


def _node_update(x, m_in, m_out, wna, wnb, wnc, bn):
    xn = jnp.dot(x, wna, preferred_element_type=F32)
    xn += jnp.dot(m_in, wnb, preferred_element_type=F32)
    xn += jax.lax.dot_general(m_out, wnc, (((0,), (0,)), ((), ())),
                              preferred_element_type=F32)
    return xn + bn


def _fused_kernel(x_ref, a_ref, e_ref,
                  wsx1_ref, wsv1_ref, bs1_ref, we1_ref, wet1_ref, w31_ref,
                  b1_ref, wna1_ref, wnb1_ref, wnc1_ref, bn1_ref,
                  wsx2_ref, wsv2_ref, bs2_ref, we2_ref, wet2_ref, w32_ref,
                  b2_ref, wna2_ref, wnb2_ref, wnc2_ref, bn2_ref,
                  wd_ref, bd_ref, out_ref,
                  e1_scr, min_scr, mout_scr, x1_scr):
    s = pl.program_id(0)

    @pl.when(s == 0)
    def _zero():
        mout_scr[...] = jnp.zeros_like(mout_scr)

    @pl.when(s < 4)
    def _layer1():
        _edge_step(s, x_ref, e_ref, a_ref, wsx1_ref[...], wsv1_ref[...],
                   bs1_ref[...], we1_ref[...], wet1_ref[...], w31_ref[...],
                   b1_ref[...], e1_scr, min_scr, mout_scr, emit_edge=True)

    @pl.when(s == 4)
    def _node1():
        x1_scr[...] = _node_update(x_ref[...], min_scr[...], mout_scr[...],
                                   wna1_ref[...], wnb1_ref[...],
                                   wnc1_ref[...], bn1_ref[...])
        mout_scr[...] = jnp.zeros_like(mout_scr)

    @pl.when(jnp.logical_and(s >= 5, s < 9))
    def _layer2():
        _edge_step(s - 5, x1_scr, e1_scr, a_ref, wsx2_ref[...], wsv2_ref[...],
                   bs2_ref[...], we2_ref[...], wet2_ref[...], w32_ref[...],
                   b2_ref[...], None, min_scr, mout_scr, emit_edge=False)

    @pl.when(s == 9)
    def _node2():
        x2 = _node_update(x1_scr[...], min_scr[...], mout_scr[...],
                          wna2_ref[...], wnb2_ref[...], wnc2_ref[...],
                          bn2_ref[...])
        out_ref[...] = jnp.dot(x2, wd_ref[...],
                               preferred_element_type=F32) + bd_ref[...]


def kernel(x, a, e, Ws1, bs1, Wai1, bai1, Wao1, bao1, Wn1, bn1, We1, be1,
           Ws2, bs2, Wai2, bai2, Wao2, bao2, Wn2, bn2, We2, be2, Wd, bd):
    f1 = x.shape[2]
    node = Wn1.shape[1]
    nlab = Wd.shape[1]
    x2d = x[0]
    a2d = a[0]
    e2d = e[0, :, :, 0]

    w31 = jnp.concatenate([Wai1.reshape(1, STACK), Wao1.reshape(1, STACK),
                           We1.reshape(1, STACK)], axis=0)
    w32 = jnp.concatenate([Wai2.reshape(1, STACK),
                           Wao2.reshape(1, STACK)], axis=0)
    b1 = jnp.stack([bai1[0], bao1[0], be1[0]]).reshape(1, 3)
    b2 = jnp.stack([bai2[0], bao2[0], bao2[0]]).reshape(1, 3)

    operands = [
        x2d, a2d, e2d,
        Ws1[:f1, :], Ws1[f1:2 * f1, :], bs1.reshape(1, STACK),
        Ws1[2 * f1:2 * f1 + 1, :], Ws1[2 * f1 + 1:, :], w31, b1,
        Wn1[:f1, :], Wn1[f1:f1 + STACK, :], Wn1[f1 + STACK:, :],
        bn1.reshape(1, node),
        Ws2[:node, :], Ws2[node:2 * node, :], bs2.reshape(1, STACK),
        Ws2[2 * node:2 * node + 1, :], Ws2[2 * node + 1:, :], w32, b2,
        Wn2[:node, :], Wn2[node:node + STACK, :], Wn2[node + STACK:, :],
        bn2.reshape(1, node),
        Wd, bd.reshape(1, nlab),
    ]
    full = lambda s: pl.BlockSpec(s, lambda i: (0,) * len(s))
    out = pl.pallas_call(
        _fused_kernel,
        grid=(10,),
        in_specs=[full(op.shape) for op in operands],
        out_shape=jax.ShapeDtypeStruct((N, nlab), F32),
        out_specs=full((N, nlab)),
        scratch_shapes=[
            pltpu.VMEM((N, N), F32),
            pltpu.VMEM((N, STACK), F32),
            pltpu.VMEM((STACK, N), F32),
            pltpu.VMEM((N, node), F32),
        ],
    )(*operands)
    return out[None, :, :]
